# baseline (speedup 1.0000x reference)
import functools

import jax
import jax.numpy as jnp
from jax import lax
from jax.experimental import pallas as pl
from jax.experimental.pallas import tpu as pltpu
from jax.experimental.pallas import tpu_sc as plsc

N_FIELDS = 26
VOCAB = 100000
DIM = 16
BATCH = 16384

ROWS = BATCH * N_FIELDS
NC, NS, L = 2, 16, 16
NW = NC * NS
ROWS_PER_W = ROWS // NW
PERIOD = 208
CHUNK = 1664
N_CHUNKS = ROWS_PER_W // CHUNK
VPC = CHUNK // L
GSZ = 128
NG = CHUNK // GSZ


def _body(tbl_hbm, x_hbm, offs_hbm, out_hbm, xv, idx2, rows, offsv, sem):
    wid = lax.axis_index("s") * NC + lax.axis_index("c")
    pltpu.sync_copy(offs_hbm, offsv)

    def chunk_step(c, _):
        base = wid * ROWS_PER_W + c * CHUNK
        pltpu.sync_copy(x_hbm.at[pl.ds(base, CHUNK)], xv)
        for j in range(VPC):
            idx2[j // (GSZ // L), pl.ds((j % (GSZ // L)) * L, L)] = (
                xv[pl.ds(j * L, L)] + offsv[pl.ds((j % 13) * L, L)]
            )
        copies = [
            pltpu.async_copy(
                tbl_hbm.at[idx2.at[g]], rows.at[pl.ds(g * GSZ, GSZ)], sem
            )
            for g in range(NG)
        ]
        for cp in copies:
            cp.wait()
        pltpu.sync_copy(rows, out_hbm.at[pl.ds(base, CHUNK)])
        return 0

    lax.fori_loop(0, N_CHUNKS, chunk_step, 0)


@functools.partial(jax.jit, static_argnums=())
def _run(tbl, xf, offs):
    mesh = plsc.VectorSubcoreMesh(core_axis_name="c", subcore_axis_name="s")
    k = functools.partial(
        pl.kernel,
        mesh=mesh,
        compiler_params=pltpu.CompilerParams(use_tc_tiling_on_sc=False),
        out_type=jax.ShapeDtypeStruct((ROWS, DIM), jnp.float32),
        scratch_types=[
            pltpu.VMEM((CHUNK,), jnp.int32),
            pltpu.VMEM((NG, GSZ), jnp.int32),
            pltpu.VMEM((CHUNK, DIM), jnp.float32),
            pltpu.VMEM((PERIOD,), jnp.int32),
            pltpu.SemaphoreType.DMA,
        ],
    )(_body)
    return k(tbl, xf, offs)


def kernel(X, tables):
    tbl = tables.reshape(N_FIELDS * VOCAB, DIM)
    xf = X.reshape(ROWS)
    offs = (jnp.arange(PERIOD, dtype=jnp.int32) % N_FIELDS) * VOCAB
    out = _run(tbl, xf, offs)
    return out.reshape(BATCH, N_FIELDS * DIM)
